# baseline (speedup 1.0000x reference)
import functools

import jax
import jax.numpy as jnp
from jax import lax
from jax.experimental import pallas as pl
from jax.experimental.pallas import tpu as pltpu
from jax.experimental.pallas import tpu_sc as plsc

_K = 1000000
_C = 16
_N = 3276800

_NC = 2
_NS = 16
_NW = _NC * _NS

_PER_W = _N // _NW
_CHUNK = 2048
_NCHUNK = _PER_W // _CHUNK
_JROWS = _CHUNK // 128


def _gather_kernel(y_hbm, table_hbm, out_hbm, idx_v, rows_v, lsem, gsem):
    wid = lax.axis_index("s") * _NC + lax.axis_index("c")
    base = wid * _PER_W
    row0 = base // 128

    def body(c, carry):
        pltpu.async_copy(
            y_hbm.at[pl.ds(pl.multiple_of(row0 + c * _JROWS, 8), _JROWS)],
            idx_v,
            lsem,
        ).wait()
        copies = []
        for j in range(_JROWS):
            copies.append(
                pltpu.async_copy(
                    table_hbm.at[idx_v.at[j]],
                    rows_v.at[pl.ds(j * 128, 128)],
                    gsem,
                )
            )
        for cp in copies:
            cp.wait()
        pltpu.async_copy(
            rows_v, out_hbm.at[pl.ds(base + c * _CHUNK, _CHUNK)], lsem
        ).wait()
        return carry

    lax.fori_loop(0, _NCHUNK, body, 0)


@jax.jit
def _run(y2d, table):
    mesh = plsc.VectorSubcoreMesh(core_axis_name="c", subcore_axis_name="s")
    return pl.kernel(
        _gather_kernel,
        out_type=jax.ShapeDtypeStruct((_N, _C), jnp.float32),
        mesh=mesh,
        scratch_types=[
            pltpu.VMEM((_JROWS, 128), jnp.int32),
            pltpu.VMEM((_CHUNK, _C), jnp.float32),
            pltpu.SemaphoreType.DMA,
            pltpu.SemaphoreType.DMA,
        ],
        compiler_params=pltpu.CompilerParams(use_tc_tiling_on_sc=False),
    )(y2d, table)


def kernel(x_labels, y_labels, emission_distr):
    y = jnp.squeeze(y_labels).astype(jnp.int32)
    y2d = y.reshape(_N // 128, 128)
    return _run(y2d, emission_distr)


# speedup vs baseline: 1.0205x; 1.0205x over previous
import functools

import jax
import jax.numpy as jnp
from jax import lax
from jax.experimental import pallas as pl
from jax.experimental.pallas import tpu as pltpu
from jax.experimental.pallas import tpu_sc as plsc

_K = 1000000
_C = 16
_N = 3276800

_NC = 2
_NS = 16
_NW = _NC * _NS

_PER_W = _N // _NW
_CHUNK = 2048
_NCHUNK = _PER_W // _CHUNK
_JROWS = _CHUNK // 128


def _gather_kernel(
    y_hbm, table_hbm, out_hbm, idx0, idx1, rows0, rows1, isem, gsem, ssem0, ssem1
):
    wid = lax.axis_index("s") * _NC + lax.axis_index("c")
    base = wid * _PER_W
    row0 = base // 128

    idx_b = (idx0, idx1)
    rows_b = (rows0, rows1)
    ssem_b = (ssem0, ssem1)

    def half(c, b):
        nb = 1 - b
        @pl.when(c >= 2)
        def _():
            pltpu.make_async_copy(
                rows_b[b], out_hbm.at[pl.ds(base, _CHUNK)], ssem_b[b]
            ).wait()

        pltpu.make_async_copy(
            y_hbm.at[pl.ds(pl.multiple_of(row0, 8), _JROWS)], idx_b[b], isem
        ).wait()
        copies = []
        for j in range(_JROWS):
            copies.append(
                pltpu.async_copy(
                    table_hbm.at[idx_b[b].at[j]],
                    rows_b[b].at[pl.ds(j * 128, 128)],
                    gsem,
                )
            )
        cc = lax.rem(c + 1, _NCHUNK)
        pltpu.make_async_copy(
            y_hbm.at[pl.ds(pl.multiple_of(row0 + cc * _JROWS, 8), _JROWS)],
            idx_b[nb],
            isem,
        ).start()
        for cp in copies:
            cp.wait()
        pltpu.make_async_copy(
            rows_b[b], out_hbm.at[pl.ds(base + c * _CHUNK, _CHUNK)], ssem_b[b]
        ).start()

    pltpu.make_async_copy(
        y_hbm.at[pl.ds(pl.multiple_of(row0, 8), _JROWS)], idx_b[0], isem
    ).start()

    def body(g, carry):
        half(2 * g, 0)
        half(2 * g + 1, 1)
        return carry

    lax.fori_loop(0, _NCHUNK // 2, body, 0)

    pltpu.make_async_copy(
        y_hbm.at[pl.ds(pl.multiple_of(row0, 8), _JROWS)], idx_b[0], isem
    ).wait()
    for b in range(2):
        pltpu.make_async_copy(
            rows_b[b], out_hbm.at[pl.ds(base, _CHUNK)], ssem_b[b]
        ).wait()


@jax.jit
def _run(y2d, table):
    mesh = plsc.VectorSubcoreMesh(core_axis_name="c", subcore_axis_name="s")
    return pl.kernel(
        _gather_kernel,
        out_type=jax.ShapeDtypeStruct((_N, _C), jnp.float32),
        mesh=mesh,
        scratch_types=[
            pltpu.VMEM((_JROWS, 128), jnp.int32),
            pltpu.VMEM((_JROWS, 128), jnp.int32),
            pltpu.VMEM((_CHUNK, _C), jnp.float32),
            pltpu.VMEM((_CHUNK, _C), jnp.float32),
            pltpu.SemaphoreType.DMA,
            pltpu.SemaphoreType.DMA,
            pltpu.SemaphoreType.DMA,
            pltpu.SemaphoreType.DMA,
        ],
        compiler_params=pltpu.CompilerParams(use_tc_tiling_on_sc=False),
    )(y2d, table)


def kernel(x_labels, y_labels, emission_distr):
    y = jnp.squeeze(y_labels).astype(jnp.int32)
    y2d = y.reshape(_N // 128, 128)
    return _run(y2d, emission_distr)


# speedup vs baseline: 1.4506x; 1.4214x over previous
import jax
import jax.numpy as jnp
from jax import lax
from jax.experimental import pallas as pl
from jax.experimental.pallas import tpu as pltpu
from jax.experimental.pallas import tpu_sc as plsc

_K = 1000000
_C = 16
_N = 3276800

_NC = 2
_NS = 16
_NW = _NC * _NS

_ABLK = _K // 128
_ABPW = -(-_ABLK // _NW)

_PER_W = _N // _NW
_BCH = 1024
_BJ = _BCH // 128
_BNCH = _PER_W // _BCH
_BROWS = _N // 128


def _format_kernel(tt_hbm, tail_hbm, scratch_hbm, abuf, arows, isem, osem):
    wid = lax.axis_index("s") * _NC + lax.axis_index("c")
    iota = lax.iota(jnp.int32, 16)

    def transpose_cols(ncols):
        def col_body(col, carry):
            v = plsc.load_gather(abuf, [iota, jnp.full((16,), col, jnp.int32)])
            plsc.store_scatter(
                arows,
                [jnp.full((16,), col // 8, jnp.int32),
                 jnp.full((16,), (col % 8) * 16, jnp.int32) + iota],
                v,
            )
            return carry
        lax.fori_loop(0, ncols, col_body, 0)

    def blk_body(blk, carry):
        pltpu.async_copy(
            tt_hbm.at[:, pl.ds(pl.multiple_of(blk * 128, 128), 128)],
            abuf, isem,
        ).wait()
        transpose_cols(128)
        pltpu.async_copy(
            scratch_hbm.at[pl.ds(pl.multiple_of(blk * 16, 8), 16), :],
            arows, osem,
        ).wait()
        return carry

    lo = wid * _ABPW
    hi = jnp.minimum(lo + _ABPW, _ABLK)
    lax.fori_loop(lo, hi, blk_body, 0)

    @pl.when(wid == _NW - 1)
    def _():
        pltpu.async_copy(tail_hbm, abuf.at[pl.ds(0, 8), :], isem).wait()
        pltpu.async_copy(
            scratch_hbm.at[pl.ds(_ABLK * 16, 8), :], abuf.at[pl.ds(0, 8), :],
            osem,
        ).wait()


def _gather_kernel(
    y_hbm, table_hbm, out_hbm,
    idx0, idx1, rows0, rows1, tb00, tb01, tb10, tb11,
    isem, gsem0, gsem1, ssem0, ssem1,
):
    wid = lax.axis_index("s") * _NC + lax.axis_index("c")
    rowbase = wid * (_PER_W // 128)
    iota = lax.iota(jnp.int32, 16)

    idx_b = (idx0, idx1)
    rows_b = (rows0, rows1)
    tb_b = ((tb00, tb01), (tb10, tb11))
    gsem_b = (gsem0, gsem1)
    ssem_b = (ssem0, ssem1)

    def idx_copy(c, buf):
        return pltpu.make_async_copy(
            y_hbm.at[pl.ds(pl.multiple_of(rowbase + c * _BJ, 8), _BJ)],
            buf, isem,
        )

    def gather_copies(par):
        return [
            pltpu.make_async_copy(
                table_hbm.at[idx_b[par].at[j]],
                rows_b[par].at[pl.ds(j * 128, 128)],
                gsem_b[par],
            )
            for j in range(_BJ)
        ]

    def store_copies(par):
        return [
            pltpu.make_async_copy(
                tb_b[par][p], out_hbm.at[p, pl.ds(0, _BJ), :], ssem_b[par]
            )
            for p in range(2)
        ]

    def transpose_chunk(rv, tbp):
        gvecs = [iota + (g * 16) for g in range(_BJ)]

        def b_body(b, cb):
            bsplat = jnp.full((16,), b, jnp.int32)
            rowvecs = [gvecs[g] + b * 128 for g in range(_BJ)]

            def s_body(s, cs):
                sbase = jnp.full((16,), s * 128, jnp.int32)
                for p in range(2):
                    cls = jnp.full((16,), s + 8 * p, jnp.int32)
                    for g in range(_BJ):
                        v = plsc.load_gather(rv, [rowvecs[g], cls])
                        plsc.store_scatter(
                            tbp[p], [bsplat, sbase + gvecs[g]], v
                        )
                return cs

            lax.fori_loop(0, 8, s_body, 0)
            return cb

        lax.fori_loop(0, _BJ, b_body, 0)

    def half(c, par):
        npar = 1 - par
        idx_copy(0, idx_b[npar]).wait()
        for cp in gather_copies(npar):
            cp.start()
        for cp in gather_copies(par):
            cp.wait()
        idx_copy(lax.rem(c + 2, _BNCH), idx_b[par]).start()

        @pl.when(c >= 2)
        def _():
            for cp in store_copies(par):
                cp.wait()

        transpose_chunk(rows_b[par], tb_b[par])
        cc0 = pl.multiple_of(rowbase + c * _BJ, 8)
        for p in range(2):
            pltpu.make_async_copy(
                tb_b[par][p], out_hbm.at[p, pl.ds(cc0, _BJ), :], ssem_b[par]
            ).start()

    idx_copy(0, idx_b[0]).start()
    idx_copy(0, idx_b[0]).wait()
    for cp in gather_copies(0):
        cp.start()
    idx_copy(1, idx_b[1]).start()

    def pair(g2, carry):
        half(2 * g2, 0)
        half(2 * g2 + 1, 1)
        return carry

    lax.fori_loop(0, _BNCH // 2, pair, 0)

    idx_copy(0, idx_b[1]).wait()
    for cp in gather_copies(0):
        cp.wait()
    for par in range(2):
        for cp in store_copies(par):
            cp.wait()


@jax.jit
def _run(y2d, tt, tail_rm):
    mesh_a = plsc.VectorSubcoreMesh(core_axis_name="c", subcore_axis_name="s")
    scratch = pl.kernel(
        _format_kernel,
        out_type=jax.ShapeDtypeStruct((_K // 8, 128), jnp.float32),
        mesh=mesh_a,
        scratch_types=[
            pltpu.VMEM((16, 128), jnp.float32),
            pltpu.VMEM((16, 128), jnp.float32),
            pltpu.SemaphoreType.DMA,
            pltpu.SemaphoreType.DMA,
        ],
        compiler_params=pltpu.CompilerParams(
            use_tc_tiling_on_sc=True, needs_layout_passes=False
        ),
    )(tt, tail_rm)
    table_rm = scratch.reshape(_K, _C)
    mesh_b = plsc.VectorSubcoreMesh(core_axis_name="c", subcore_axis_name="s")
    out3 = pl.kernel(
        _gather_kernel,
        out_type=jax.ShapeDtypeStruct((2, _BROWS, 1024), jnp.float32),
        mesh=mesh_b,
        scratch_types=[
            pltpu.VMEM((_BJ, 128), jnp.int32),
            pltpu.VMEM((_BJ, 128), jnp.int32),
            pltpu.VMEM((_BCH, _C), jnp.float32),
            pltpu.VMEM((_BCH, _C), jnp.float32),
            pltpu.VMEM((_BJ, 1024), jnp.float32),
            pltpu.VMEM((_BJ, 1024), jnp.float32),
            pltpu.VMEM((_BJ, 1024), jnp.float32),
            pltpu.VMEM((_BJ, 1024), jnp.float32),
            pltpu.SemaphoreType.DMA,
            pltpu.SemaphoreType.DMA,
            pltpu.SemaphoreType.DMA,
            pltpu.SemaphoreType.DMA,
            pltpu.SemaphoreType.DMA,
        ],
        compiler_params=pltpu.CompilerParams(
            use_tc_tiling_on_sc=False, needs_layout_passes=False
        ),
    )(y2d, table_rm)
    out4 = out3.reshape(2, _BROWS, 8, 128)
    return out4.transpose(1, 3, 0, 2).reshape(_N, _C)


def kernel(x_labels, y_labels, emission_distr):
    y = jnp.squeeze(y_labels).astype(jnp.int32)
    y2d = y.reshape(_BROWS, 128)
    tt = emission_distr.T
    tail_rm = emission_distr[_ABLK * 128:].reshape(8, 128)
    return _run(y2d, tt, tail_rm)


# speedup vs baseline: 1.8406x; 1.2689x over previous
import jax
import jax.numpy as jnp
from jax import lax
from jax.experimental import pallas as pl
from jax.experimental.pallas import tpu as pltpu
from jax.experimental.pallas import tpu_sc as plsc

_K = 1000000
_C = 16
_N = 3276800

_NC = 2
_NS = 16
_NW = _NC * _NS

_ABLK = _K // 128
_ABPW = -(-_ABLK // _NW)

_PER_W = _N // _NW
_BCH = 1024
_BJ = _BCH // 128
_BNCH = _PER_W // _BCH
_BROWS = _N // 128


def _format_kernel(
    tt_hbm, tail_hbm, scratch_hbm,
    abuf0, abuf1, arows0, arows1, isem0, isem1, osem0, osem1,
):
    wid = lax.axis_index("s") * _NC + lax.axis_index("c")
    iota = lax.iota(jnp.int32, 16)
    ab = (abuf0, abuf1)
    ar = (arows0, arows1)
    isem = (isem0, isem1)
    osem = (osem0, osem1)
    lo = wid * _ABPW
    clamp = _ABLK - 1

    def in_copy(k, par):
        blk = jnp.minimum(lo + k, clamp)
        return pltpu.make_async_copy(
            tt_hbm.at[:, pl.ds(pl.multiple_of(blk * 128, 128), 128)],
            ab[par], isem[par],
        )

    def out_copy(k, par):
        blk = jnp.minimum(lo + k, clamp)
        return pltpu.make_async_copy(
            ar[par],
            scratch_hbm.at[pl.ds(pl.multiple_of(blk * 16, 8), 16), :],
            osem[par],
        )

    def transpose_block(src, dst):
        svecs = [jnp.full((16,), r * 16, jnp.int32) + iota for r in range(8)]

        def grp_body(c8, carry):
            rowsplat = jnp.full((16,), c8, jnp.int32)
            for r in range(8):
                v = plsc.load_gather(
                    src, [iota, jnp.full((16,), c8 * 8 + r, jnp.int32)]
                )
                plsc.store_scatter(dst, [rowsplat, svecs[r]], v)
            return carry

        lax.fori_loop(0, 16, grp_body, 0)

    def step(k, par):
        in_copy(k + 1, 1 - par).start()
        in_copy(k, par).wait()

        @pl.when(k >= 2)
        def _():
            out_copy(0, par).wait()

        transpose_block(ab[par], ar[par])
        out_copy(k, par).start()

    in_copy(0, 0).start()

    def pair(m, carry):
        step(2 * m, 0)
        step(2 * m + 1, 1)
        return carry

    lax.fori_loop(0, _ABPW // 2, pair, 0)

    in_copy(_ABPW - 1, 0).wait()
    out_copy(0, 0).wait()
    transpose_block(ab[0], ar[0])
    out_copy(_ABPW - 1, 0).start()

    out_copy(0, 0).wait()
    out_copy(0, 1).wait()

    @pl.when(wid == _NW - 1)
    def _():
        pltpu.async_copy(tail_hbm, ab[0].at[pl.ds(0, 8), :], isem[0]).wait()
        pltpu.async_copy(
            ab[0].at[pl.ds(0, 8), :], scratch_hbm.at[pl.ds(_ABLK * 16, 8), :],
            osem[0],
        ).wait()


def _gather_kernel(
    y_hbm, table_hbm, out_hbm,
    idx0, idx1, rows0, rows1, tb00, tb01, tb10, tb11,
    isem, gsem0, gsem1, ssem0, ssem1,
):
    wid = lax.axis_index("s") * _NC + lax.axis_index("c")
    rowbase = wid * (_PER_W // 128)
    iota = lax.iota(jnp.int32, 16)

    idx_b = (idx0, idx1)
    rows_b = (rows0, rows1)
    tb_b = ((tb00, tb01), (tb10, tb11))
    gsem_b = (gsem0, gsem1)
    ssem_b = (ssem0, ssem1)

    def idx_copy(c, buf):
        return pltpu.make_async_copy(
            y_hbm.at[pl.ds(pl.multiple_of(rowbase + c * _BJ, 8), _BJ)],
            buf, isem,
        )

    def gather_copies(par):
        return [
            pltpu.make_async_copy(
                table_hbm.at[idx_b[par].at[j]],
                rows_b[par].at[pl.ds(j * 128, 128)],
                gsem_b[par],
            )
            for j in range(_BJ)
        ]

    def store_copies(par):
        return [
            pltpu.make_async_copy(
                tb_b[par][p], out_hbm.at[p, pl.ds(0, _BJ), :], ssem_b[par]
            )
            for p in range(2)
        ]

    def transpose_chunk(rv, tbp):
        gvecs = [iota + (g * 16) for g in range(_BJ)]

        def b_body(b, cb):
            bsplat = jnp.full((16,), b, jnp.int32)
            rowvecs = [gvecs[g] + b * 128 for g in range(_BJ)]

            def s_body(s, cs):
                sbase = jnp.full((16,), s * 128, jnp.int32)
                for p in range(2):
                    cls = jnp.full((16,), s + 8 * p, jnp.int32)
                    for g in range(_BJ):
                        v = plsc.load_gather(rv, [rowvecs[g], cls])
                        plsc.store_scatter(
                            tbp[p], [bsplat, sbase + gvecs[g]], v
                        )
                return cs

            lax.fori_loop(0, 8, s_body, 0)
            return cb

        lax.fori_loop(0, _BJ, b_body, 0)

    def half(c, par):
        npar = 1 - par
        idx_copy(0, idx_b[npar]).wait()
        for cp in gather_copies(npar):
            cp.start()
        for cp in gather_copies(par):
            cp.wait()
        idx_copy(lax.rem(c + 2, _BNCH), idx_b[par]).start()

        @pl.when(c >= 2)
        def _():
            for cp in store_copies(par):
                cp.wait()

        transpose_chunk(rows_b[par], tb_b[par])
        cc0 = pl.multiple_of(rowbase + c * _BJ, 8)
        for p in range(2):
            pltpu.make_async_copy(
                tb_b[par][p], out_hbm.at[p, pl.ds(cc0, _BJ), :], ssem_b[par]
            ).start()

    idx_copy(0, idx_b[0]).start()
    idx_copy(0, idx_b[0]).wait()
    for cp in gather_copies(0):
        cp.start()
    idx_copy(1, idx_b[1]).start()

    def pair(g2, carry):
        half(2 * g2, 0)
        half(2 * g2 + 1, 1)
        return carry

    lax.fori_loop(0, _BNCH // 2, pair, 0)

    idx_copy(0, idx_b[1]).wait()
    for cp in gather_copies(0):
        cp.wait()
    for par in range(2):
        for cp in store_copies(par):
            cp.wait()


@jax.jit
def _run(y2d, tt, tail_rm):
    mesh_a = plsc.VectorSubcoreMesh(core_axis_name="c", subcore_axis_name="s")
    scratch = pl.kernel(
        _format_kernel,
        out_type=jax.ShapeDtypeStruct((_K // 8, 128), jnp.float32),
        mesh=mesh_a,
        scratch_types=[
            pltpu.VMEM((16, 128), jnp.float32),
            pltpu.VMEM((16, 128), jnp.float32),
            pltpu.VMEM((16, 128), jnp.float32),
            pltpu.VMEM((16, 128), jnp.float32),
            pltpu.SemaphoreType.DMA,
            pltpu.SemaphoreType.DMA,
            pltpu.SemaphoreType.DMA,
            pltpu.SemaphoreType.DMA,
        ],
        compiler_params=pltpu.CompilerParams(
            use_tc_tiling_on_sc=True, needs_layout_passes=False
        ),
    )(tt, tail_rm)
    table_rm = scratch.reshape(_K, _C)
    mesh_b = plsc.VectorSubcoreMesh(core_axis_name="c", subcore_axis_name="s")
    out3 = pl.kernel(
        _gather_kernel,
        out_type=jax.ShapeDtypeStruct((2, _BROWS, 1024), jnp.float32),
        mesh=mesh_b,
        scratch_types=[
            pltpu.VMEM((_BJ, 128), jnp.int32),
            pltpu.VMEM((_BJ, 128), jnp.int32),
            pltpu.VMEM((_BCH, _C), jnp.float32),
            pltpu.VMEM((_BCH, _C), jnp.float32),
            pltpu.VMEM((_BJ, 1024), jnp.float32),
            pltpu.VMEM((_BJ, 1024), jnp.float32),
            pltpu.VMEM((_BJ, 1024), jnp.float32),
            pltpu.VMEM((_BJ, 1024), jnp.float32),
            pltpu.SemaphoreType.DMA,
            pltpu.SemaphoreType.DMA,
            pltpu.SemaphoreType.DMA,
            pltpu.SemaphoreType.DMA,
            pltpu.SemaphoreType.DMA,
        ],
        compiler_params=pltpu.CompilerParams(
            use_tc_tiling_on_sc=False, needs_layout_passes=False
        ),
    )(y2d, table_rm)
    out4 = out3.reshape(2, _BROWS, 8, 128)
    return out4.transpose(1, 3, 0, 2).reshape(_N, _C)


def kernel(x_labels, y_labels, emission_distr):
    y = jnp.squeeze(y_labels).astype(jnp.int32)
    y2d = y.reshape(_BROWS, 128)
    tt = emission_distr.T
    tail_rm = emission_distr[_ABLK * 128:].reshape(8, 128)
    return _run(y2d, tt, tail_rm)
